# baseline (device time: 129166 ns/iter reference)
import numpy as np
import jax
import jax.numpy as jnp
from jax import lax
from jax.experimental import pallas as pl
from jax.experimental.pallas import tpu as pltpu

N_DEV = 4


def _rope_tables(Sq, Dh, Hl, Bl):
    inv = 1.0 / (10000.0 ** (np.arange(0, Dh, 2) / Dh))
    pos = np.arange(Sq)[:, None] * inv[None, :]
    cos = np.repeat(np.cos(pos), 2, axis=-1).astype(np.float32)
    sin = np.repeat(np.sin(pos), 2, axis=-1).astype(np.float32)
    cos_t = np.tile(np.tile(cos, (1, Hl)), (Bl, 1))
    sin_t = np.tile(np.tile(sin, (1, Hl)), (Bl, 1))
    P = np.zeros((Hl * Dh, Hl * Dh), np.float32)
    for h in range(Hl):
        o = h * Dh
        for k in range(Dh // 2):
            P[o + 2 * k + 1, o + 2 * k] = -1.0
            P[o + 2 * k, o + 2 * k + 1] = 1.0
    return cos_t, sin_t, P


def kernel(x, Wq, Wk, Wv, Wo):
    Bl, Sq, D = x.shape
    HD = Wq.shape[1]
    Dh = 64
    Hl = HD // Dh
    cos_np, sin_np, P_np = _rope_tables(Sq, Dh, Hl, Bl)
    cos_t = jnp.asarray(cos_np)
    sin_t = jnp.asarray(sin_np)
    P_m = jnp.asarray(P_np)

    def body(x_ref, wq_ref, wk_ref, wv_ref, wo_ref, cos_ref, sin_ref, p_ref,
             out_ref, xg_ref, partial_ref, rs_ref, ctx_ref,
             ag_send, ag_recv, rs_send, rs_recv):
        me = lax.axis_index("i")
        left = lax.rem(me + N_DEV - 1, N_DEV)
        right = lax.rem(me + 1, N_DEV)

        barrier = pltpu.get_barrier_semaphore()
        for nbr in (left, right):
            pl.semaphore_signal(
                barrier, inc=1,
                device_id=(nbr,), device_id_type=pl.DeviceIdType.MESH,
            )
        pl.semaphore_wait(barrier, 2)

        wq = wq_ref[...]
        wk = wk_ref[...]
        wv = wv_ref[...]
        wo = wo_ref[...]
        cos2 = cos_ref[...]
        sin2 = sin_ref[...]
        pm = p_ref[...]

        def compute_chunk(xc, t):
            x2 = xc.reshape(Bl * Sq, D)
            q = jnp.dot(x2, wq, preferred_element_type=jnp.float32)
            k = jnp.dot(x2, wk, preferred_element_type=jnp.float32)
            v = jnp.dot(x2, wv, preferred_element_type=jnp.float32)
            q = q * cos2 + jnp.dot(q, pm, preferred_element_type=jnp.float32) * sin2
            k = k * cos2 + jnp.dot(k, pm, preferred_element_type=jnp.float32) * sin2
            for b in range(Bl):
                qb = q[b * Sq:(b + 1) * Sq]
                kb = k[b * Sq:(b + 1) * Sq]
                vb = v[b * Sq:(b + 1) * Sq]
                for h in range(Hl):
                    qh = qb[:, h * Dh:(h + 1) * Dh]
                    kh = kb[:, h * Dh:(h + 1) * Dh]
                    vh = vb[:, h * Dh:(h + 1) * Dh]
                    s = lax.dot_general(
                        qh, kh, (((1,), (1,)), ((), ())),
                        preferred_element_type=jnp.float32,
                    ) * 0.125
                    m = jnp.max(s, axis=-1, keepdims=True)
                    w = jnp.exp(s - m)
                    w = w / jnp.sum(w, axis=-1, keepdims=True)
                    ctx_ref[b, :, h * Dh:(h + 1) * Dh] = jnp.dot(
                        w, vh, preferred_element_type=jnp.float32
                    )
            c2 = ctx_ref[...].reshape(Bl * Sq, HD)
            po = jnp.dot(c2, wo, preferred_element_type=jnp.float32)
            partial_ref[t] = po.reshape(Bl, Sq, D)

        rdma0 = pltpu.make_async_remote_copy(
            src_ref=x_ref, dst_ref=xg_ref.at[0],
            send_sem=ag_send.at[0], recv_sem=ag_recv.at[0],
            device_id=(right,), device_id_type=pl.DeviceIdType.MESH,
        )
        rdma0.start()
        compute_chunk(x_ref[...], 3)
        rdma0.wait()
        for u in (1, 2):
            rdma = pltpu.make_async_remote_copy(
                src_ref=xg_ref.at[u - 1], dst_ref=xg_ref.at[u],
                send_sem=ag_send.at[u], recv_sem=ag_recv.at[u],
                device_id=(right,), device_id_type=pl.DeviceIdType.MESH,
            )
            rdma.start()
            compute_chunk(xg_ref[u - 1], u - 1)
            rdma.wait()
        compute_chunk(xg_ref[2], 2)

        for s in range(N_DEV - 1):
            rdma = pltpu.make_async_remote_copy(
                src_ref=partial_ref.at[s], dst_ref=rs_ref.at[s],
                send_sem=rs_send.at[s], recv_sem=rs_recv.at[s],
                device_id=(right,), device_id_type=pl.DeviceIdType.MESH,
            )
            rdma.start()
            rdma.wait()
            if s < N_DEV - 2:
                partial_ref[s + 1] = partial_ref[s + 1] + rs_ref[s]
            else:
                out_ref[...] = partial_ref[N_DEV - 1] + rs_ref[s]

    return pl.pallas_call(
        body,
        out_shape=jax.ShapeDtypeStruct((Bl, Sq, D), jnp.float32),
        in_specs=[pl.BlockSpec(memory_space=pltpu.VMEM)] * 8,
        out_specs=pl.BlockSpec(memory_space=pltpu.VMEM),
        scratch_shapes=[
            pltpu.VMEM((N_DEV - 1, Bl, Sq, D), jnp.float32),
            pltpu.VMEM((N_DEV, Bl, Sq, D), jnp.float32),
            pltpu.VMEM((N_DEV - 1, Bl, Sq, D), jnp.float32),
            pltpu.VMEM((Bl, Sq, HD), jnp.float32),
            pltpu.SemaphoreType.DMA((N_DEV - 1,)),
            pltpu.SemaphoreType.DMA((N_DEV - 1,)),
            pltpu.SemaphoreType.DMA((N_DEV - 1,)),
            pltpu.SemaphoreType.DMA((N_DEV - 1,)),
        ],
        compiler_params=pltpu.CompilerParams(collective_id=0),
    )(x, Wq, Wk, Wv, Wo, cos_t, sin_t, P_m)


# device time: 68700 ns/iter; 1.8801x vs baseline; 1.8801x over previous
import numpy as np
import jax
import jax.numpy as jnp
from jax import lax
from jax.experimental import pallas as pl
from jax.experimental.pallas import tpu as pltpu

N_DEV = 4


def _rope_tables(Sq, Dh, n_heads, Bl):
    inv = 1.0 / (10000.0 ** (np.arange(0, Dh, 2) / Dh))
    pos = np.arange(Sq)[:, None] * inv[None, :]
    cos = np.repeat(np.cos(pos), 2, axis=-1).astype(np.float32)
    sin = np.repeat(np.sin(pos), 2, axis=-1).astype(np.float32)
    cos_t = np.tile(np.tile(cos, (1, n_heads)), (Bl, 1))
    sin_t = np.tile(np.tile(sin, (1, n_heads)), (Bl, 1))
    n = n_heads * Dh
    P = np.zeros((n, n), np.float32)
    for h in range(n_heads):
        o = h * Dh
        for k in range(Dh // 2):
            P[o + 2 * k + 1, o + 2 * k] = -1.0
            P[o + 2 * k, o + 2 * k + 1] = 1.0
    return cos_t, sin_t, P


def kernel(x, Wq, Wk, Wv, Wo):
    Bl, Sq, D = x.shape
    HD = Wq.shape[1]
    Dh = 64
    Hh = (HD // Dh) // 2
    HW = HD // 2
    BS = Bl * Sq

    wcat = jnp.concatenate([Wq, Wk, Wv, Wo.T], axis=0)
    cos_np, sin_np, P_np = _rope_tables(Sq, Dh, Hh, Bl)
    cos_t = jnp.asarray(cos_np)
    sin_t = jnp.asarray(sin_np)
    P_m = jnp.asarray(P_np)

    def body(x_ref, wcat_ref, cos_ref, sin_ref, p_ref, out_ref,
             wcw_ref, wccw_ref, ctx_ref,
             cw_send, cw_recv, ccw_send, ccw_recv):
        me = lax.axis_index("i")
        left = lax.rem(me + N_DEV - 1, N_DEV)
        right = lax.rem(me + 1, N_DEV)

        barrier = pltpu.get_barrier_semaphore()
        for nbr in (left, right):
            pl.semaphore_signal(
                barrier, inc=1,
                device_id=(nbr,), device_id_type=pl.DeviceIdType.MESH,
            )
        pl.semaphore_wait(barrier, 2)

        x2 = x_ref[...].reshape(BS, D)
        cos2 = cos_ref[...]
        sin2 = sin_ref[...]
        pm = p_ref[...]

        def compute_block(wblk):
            q = jnp.dot(x2, wblk[0:D], preferred_element_type=jnp.float32)
            k = jnp.dot(x2, wblk[D:2 * D], preferred_element_type=jnp.float32)
            v = jnp.dot(x2, wblk[2 * D:3 * D], preferred_element_type=jnp.float32)
            q = q * cos2 + jnp.dot(q, pm, preferred_element_type=jnp.float32) * sin2
            k = k * cos2 + jnp.dot(k, pm, preferred_element_type=jnp.float32) * sin2
            for b in range(Bl):
                for h in range(Hh):
                    qh = q[b * Sq:(b + 1) * Sq, h * Dh:(h + 1) * Dh]
                    kh = k[b * Sq:(b + 1) * Sq, h * Dh:(h + 1) * Dh]
                    vh = v[b * Sq:(b + 1) * Sq, h * Dh:(h + 1) * Dh]
                    s = lax.dot_general(
                        qh, kh, (((1,), (1,)), ((), ())),
                        preferred_element_type=jnp.float32,
                    ) * 0.125
                    m = jnp.max(s, axis=-1, keepdims=True)
                    w = jnp.exp(s - m)
                    w = w / jnp.sum(w, axis=-1, keepdims=True)
                    ctx_ref[b * Sq:(b + 1) * Sq, h * Dh:(h + 1) * Dh] = jnp.dot(
                        w, vh, preferred_element_type=jnp.float32
                    )
            return lax.dot_general(
                ctx_ref[...], wblk[3 * D:4 * D], (((1,), (1,)), ((), ())),
                preferred_element_type=jnp.float32,
            )

        def make_hop(u):
            if u == 0:
                cw_src = wcat_ref.at[:, 0:HW]
                ccw_src = wcat_ref.at[:, HW:HD]
            else:
                cw_src = wcw_ref.at[u - 1]
                ccw_src = wccw_ref.at[u - 1]
            cw = pltpu.make_async_remote_copy(
                src_ref=cw_src, dst_ref=wcw_ref.at[u],
                send_sem=cw_send.at[u], recv_sem=cw_recv.at[u],
                device_id=(right,), device_id_type=pl.DeviceIdType.MESH,
            )
            ccw = pltpu.make_async_remote_copy(
                src_ref=ccw_src, dst_ref=wccw_ref.at[u],
                send_sem=ccw_send.at[u], recv_sem=ccw_recv.at[u],
                device_id=(left,), device_id_type=pl.DeviceIdType.MESH,
            )
            cw.start()
            ccw.start()
            return cw, ccw

        cw, ccw = make_hop(0)
        acc = compute_block(wcat_ref[:, 0:HW])
        acc = acc + compute_block(wcat_ref[:, HW:HD])
        cw.wait()
        ccw.wait()
        for u in range(1, N_DEV - 1):
            cw, ccw = make_hop(u)
            acc = acc + compute_block(wcw_ref[u - 1])
            acc = acc + compute_block(wccw_ref[u - 1])
            cw.wait()
            ccw.wait()
        acc = acc + compute_block(wcw_ref[N_DEV - 2])
        acc = acc + compute_block(wccw_ref[N_DEV - 2])
        out_ref[...] = acc.reshape(Bl, Sq, D)

    return pl.pallas_call(
        body,
        out_shape=jax.ShapeDtypeStruct((Bl, Sq, D), jnp.float32),
        in_specs=[pl.BlockSpec(memory_space=pltpu.VMEM)] * 5,
        out_specs=pl.BlockSpec(memory_space=pltpu.VMEM),
        scratch_shapes=[
            pltpu.VMEM((N_DEV - 1, 4 * D, HW), jnp.float32),
            pltpu.VMEM((N_DEV - 1, 4 * D, HW), jnp.float32),
            pltpu.VMEM((BS, HW), jnp.float32),
            pltpu.SemaphoreType.DMA((N_DEV - 1,)),
            pltpu.SemaphoreType.DMA((N_DEV - 1,)),
            pltpu.SemaphoreType.DMA((N_DEV - 1,)),
            pltpu.SemaphoreType.DMA((N_DEV - 1,)),
        ],
        compiler_params=pltpu.CompilerParams(collective_id=0),
    )(x, wcat, cos_t, sin_t, P_m)


# device time: 42913 ns/iter; 3.0100x vs baseline; 1.6009x over previous
import numpy as np
import jax
import jax.numpy as jnp
from jax import lax
from jax.experimental import pallas as pl
from jax.experimental.pallas import tpu as pltpu

N_DEV = 4


def _rope_tables(Sq, Dh, n_heads, Bl):
    inv = 1.0 / (10000.0 ** (np.arange(0, Dh, 2) / Dh))
    pos = np.arange(Sq)[:, None] * inv[None, :]
    cos = np.repeat(np.cos(pos), 2, axis=-1).astype(np.float32)
    sin = np.repeat(np.sin(pos), 2, axis=-1).astype(np.float32)
    cos_t = np.tile(np.tile(cos, (1, n_heads)), (Bl, 1))
    sin_t = np.tile(np.tile(sin, (1, n_heads)), (Bl, 1))
    n = n_heads * Dh
    P = np.zeros((n, n), np.float32)
    for h in range(n_heads):
        o = h * Dh
        for k in range(Dh // 2):
            P[o + 2 * k + 1, o + 2 * k] = -1.0
            P[o + 2 * k, o + 2 * k + 1] = 1.0
    return cos_t, sin_t, P


def kernel(x, Wq, Wk, Wv, Wo):
    Bl, Sq, D = x.shape
    HD = Wq.shape[1]
    Dh = 64
    Hh = (HD // Dh) // 2
    HW = HD // 2
    BS = Bl * Sq

    wcat = jnp.concatenate([Wq, Wk, Wv, Wo.T], axis=0).astype(jnp.bfloat16)
    cos_np, sin_np, P_np = _rope_tables(Sq, Dh, Hh, Bl)
    cos_t = jnp.asarray(cos_np)
    sin_t = jnp.asarray(sin_np)
    P_m = jnp.asarray(P_np).astype(jnp.bfloat16)

    def body(x_ref, wcat_ref, cos_ref, sin_ref, p_ref, out_ref,
             wcw_ref, wccw_ref, ctx_ref,
             cw_send, cw_recv, ccw_send, ccw_recv):
        me = lax.axis_index("i")
        left = lax.rem(me + N_DEV - 1, N_DEV)
        right = lax.rem(me + 1, N_DEV)

        barrier = pltpu.get_barrier_semaphore()
        for nbr in (left, right):
            pl.semaphore_signal(
                barrier, inc=1,
                device_id=(nbr,), device_id_type=pl.DeviceIdType.MESH,
            )
        pl.semaphore_wait(barrier, 2)

        x2 = x_ref[...].reshape(BS, D).astype(jnp.bfloat16)
        cos2 = cos_ref[...]
        sin2 = sin_ref[...]
        pm = p_ref[...]

        def compute_block(wblk):
            q = jnp.dot(x2, wblk[0:D], preferred_element_type=jnp.float32)
            k = jnp.dot(x2, wblk[D:2 * D], preferred_element_type=jnp.float32)
            v = jnp.dot(
                x2, wblk[2 * D:3 * D], preferred_element_type=jnp.float32
            ).astype(jnp.bfloat16)
            qp = jnp.dot(q.astype(jnp.bfloat16), pm, preferred_element_type=jnp.float32)
            kp = jnp.dot(k.astype(jnp.bfloat16), pm, preferred_element_type=jnp.float32)
            q = (q * cos2 + qp * sin2).astype(jnp.bfloat16)
            k = (k * cos2 + kp * sin2).astype(jnp.bfloat16)
            for b in range(Bl):
                for h in range(Hh):
                    qh = q[b * Sq:(b + 1) * Sq, h * Dh:(h + 1) * Dh]
                    kh = k[b * Sq:(b + 1) * Sq, h * Dh:(h + 1) * Dh]
                    vh = v[b * Sq:(b + 1) * Sq, h * Dh:(h + 1) * Dh]
                    s = lax.dot_general(
                        qh, kh, (((1,), (1,)), ((), ())),
                        preferred_element_type=jnp.float32,
                    ) * 0.125
                    m = jnp.max(s, axis=-1, keepdims=True)
                    w = jnp.exp(s - m)
                    w = (w / jnp.sum(w, axis=-1, keepdims=True)).astype(jnp.bfloat16)
                    ctx_ref[b * Sq:(b + 1) * Sq, h * Dh:(h + 1) * Dh] = jnp.dot(
                        w, vh, preferred_element_type=jnp.float32
                    ).astype(jnp.bfloat16)
            return lax.dot_general(
                ctx_ref[...], wblk[3 * D:4 * D], (((1,), (1,)), ((), ())),
                preferred_element_type=jnp.float32,
            )

        def make_hop(u):
            if u == 0:
                cw_src = wcat_ref.at[:, 0:HW]
                ccw_src = wcat_ref.at[:, HW:HD]
            else:
                cw_src = wcw_ref.at[u - 1]
                ccw_src = wccw_ref.at[u - 1]
            cw = pltpu.make_async_remote_copy(
                src_ref=cw_src, dst_ref=wcw_ref.at[u],
                send_sem=cw_send.at[u], recv_sem=cw_recv.at[u],
                device_id=(right,), device_id_type=pl.DeviceIdType.MESH,
            )
            ccw = pltpu.make_async_remote_copy(
                src_ref=ccw_src, dst_ref=wccw_ref.at[u],
                send_sem=ccw_send.at[u], recv_sem=ccw_recv.at[u],
                device_id=(left,), device_id_type=pl.DeviceIdType.MESH,
            )
            cw.start()
            ccw.start()
            return cw, ccw

        cw, ccw = make_hop(0)
        acc = compute_block(wcat_ref[:, 0:HW])
        acc = acc + compute_block(wcat_ref[:, HW:HD])
        cw.wait()
        ccw.wait()
        for u in range(1, N_DEV - 1):
            cw, ccw = make_hop(u)
            acc = acc + compute_block(wcw_ref[u - 1])
            acc = acc + compute_block(wccw_ref[u - 1])
            cw.wait()
            ccw.wait()
        acc = acc + compute_block(wcw_ref[N_DEV - 2])
        acc = acc + compute_block(wccw_ref[N_DEV - 2])
        out_ref[...] = acc.reshape(Bl, Sq, D)

    return pl.pallas_call(
        body,
        out_shape=jax.ShapeDtypeStruct((Bl, Sq, D), jnp.float32),
        in_specs=[pl.BlockSpec(memory_space=pltpu.VMEM)] * 5,
        out_specs=pl.BlockSpec(memory_space=pltpu.VMEM),
        scratch_shapes=[
            pltpu.VMEM((N_DEV - 1, 4 * D, HW), jnp.bfloat16),
            pltpu.VMEM((N_DEV - 1, 4 * D, HW), jnp.bfloat16),
            pltpu.VMEM((BS, HW), jnp.bfloat16),
            pltpu.SemaphoreType.DMA((N_DEV - 1,)),
            pltpu.SemaphoreType.DMA((N_DEV - 1,)),
            pltpu.SemaphoreType.DMA((N_DEV - 1,)),
            pltpu.SemaphoreType.DMA((N_DEV - 1,)),
        ],
        compiler_params=pltpu.CompilerParams(collective_id=0),
    )(x, wcat, cos_t, sin_t, P_m)


# device time: 21633 ns/iter; 5.9708x vs baseline; 1.9837x over previous
import numpy as np
import jax
import jax.numpy as jnp
from jax import lax
from jax.experimental import pallas as pl
from jax.experimental.pallas import tpu as pltpu

N_DEV = 4


def _rope_tables(Sq, Dh, n_heads, Bl):
    inv = 1.0 / (10000.0 ** (np.arange(0, Dh, 2) / Dh))
    pos = np.arange(Sq)[:, None] * inv[None, :]
    cos = np.repeat(np.cos(pos), 2, axis=-1).astype(np.float32)
    sin = np.repeat(np.sin(pos), 2, axis=-1).astype(np.float32)
    cos_t = np.tile(np.tile(cos, (1, n_heads)), (Bl, 1))
    sin_t = np.tile(np.tile(sin, (1, n_heads)), (Bl, 1))
    n = n_heads * Dh
    P = np.zeros((n, n), np.float32)
    for h in range(n_heads):
        o = h * Dh
        for k in range(Dh // 2):
            P[o + 2 * k + 1, o + 2 * k] = -1.0
            P[o + 2 * k, o + 2 * k + 1] = 1.0
    return cos_t, sin_t, P


def kernel(x, Wq, Wk, Wv, Wo):
    Bl, Sq, D = x.shape
    HD = Wq.shape[1]
    Dh = 64
    HW = HD // 2
    BS = Bl * Sq
    NH = N_DEV - 1

    wcat = jnp.concatenate([Wq, Wk, Wv, Wo.T], axis=0).astype(jnp.bfloat16)
    cos_np, sin_np, P_np = _rope_tables(Sq, Dh, HD // Dh, Bl)
    cos_t = jnp.asarray(cos_np)
    sin_t = jnp.asarray(sin_np)
    P_m = jnp.asarray(P_np).astype(jnp.bfloat16)

    def body(x_ref, wcat_ref, cos_ref, sin_ref, p_ref, out_ref,
             wcw_ref, wccw_ref, ctx_ref,
             cw_send, cw_recv, ccw_send, ccw_recv):
        me = lax.axis_index("i")
        left = lax.rem(me + N_DEV - 1, N_DEV)
        right = lax.rem(me + 1, N_DEV)

        barrier = pltpu.get_barrier_semaphore()
        for nbr in (left, right):
            pl.semaphore_signal(
                barrier, inc=1,
                device_id=(nbr,), device_id_type=pl.DeviceIdType.MESH,
            )
        pl.semaphore_wait(barrier, 2)

        RC = 2 * D

        def make_chunk(u, c, ccw):
            w_ref = wccw_ref if ccw else wcw_ref
            lo = c * RC
            if u == 0:
                cols = (HW, HD) if ccw else (0, HW)
                src = wcat_ref.at[lo:lo + RC, cols[0]:cols[1]]
            else:
                src = w_ref.at[u - 1, lo:lo + RC, :]
            sems = (ccw_send, ccw_recv) if ccw else (cw_send, cw_recv)
            tgt = left if ccw else right
            r = pltpu.make_async_remote_copy(
                src_ref=src, dst_ref=w_ref.at[u, lo:lo + RC, :],
                send_sem=sems[0].at[u, c], recv_sem=sems[1].at[u, c],
                device_id=(tgt,), device_id_type=pl.DeviceIdType.MESH,
            )
            r.start()
            return r

        rd = {}
        for c in range(2):
            rd["cw", 0, c] = make_chunk(0, c, ccw=False)
            rd["ccw", 0, c] = make_chunk(0, c, ccw=True)

        x2 = x_ref[...].reshape(BS, D).astype(jnp.bfloat16)
        cos2 = cos_ref[...]
        sin2 = sin_ref[...]
        pm = p_ref[...]

        def compute_block(wblk):
            width = wblk.shape[1]
            cw_ = cos2[:, :width]
            sw_ = sin2[:, :width]
            pw_ = pm[:width, :width]
            q = jnp.dot(x2, wblk[0:D], preferred_element_type=jnp.float32)
            k = jnp.dot(x2, wblk[D:2 * D], preferred_element_type=jnp.float32)
            v = jnp.dot(
                x2, wblk[2 * D:3 * D], preferred_element_type=jnp.float32
            ).astype(jnp.bfloat16)
            qp = jnp.dot(q.astype(jnp.bfloat16), pw_, preferred_element_type=jnp.float32)
            kp = jnp.dot(k.astype(jnp.bfloat16), pw_, preferred_element_type=jnp.float32)
            q = (q * cw_ + qp * sw_).astype(jnp.bfloat16)
            k = (k * cw_ + kp * sw_).astype(jnp.bfloat16)
            po = None
            for b in range(Bl):
                for h in range(width // Dh):
                    qh = q[b * Sq:(b + 1) * Sq, h * Dh:(h + 1) * Dh]
                    kh = k[b * Sq:(b + 1) * Sq, h * Dh:(h + 1) * Dh]
                    vh = v[b * Sq:(b + 1) * Sq, h * Dh:(h + 1) * Dh]
                    s = lax.dot_general(
                        qh, kh, (((1,), (1,)), ((), ())),
                        preferred_element_type=jnp.float32,
                    ) * 0.125
                    w = jnp.exp(s)
                    denom = jnp.sum(w, axis=-1, keepdims=True)
                    cx = jnp.dot(
                        w.astype(jnp.bfloat16), vh,
                        preferred_element_type=jnp.float32,
                    ) / denom
                    ctx_ref[b * Sq:(b + 1) * Sq, h * Dh:(h + 1) * Dh] = (
                        cx.astype(jnp.bfloat16)
                    )
            return lax.dot_general(
                ctx_ref[:, :width], wblk[3 * D:4 * D],
                (((1,), (1,)), ((), ())),
                preferred_element_type=jnp.float32,
            )

        acc = compute_block(wcat_ref[...])

        for c in range(2):
            rd["cw", 0, c].wait_recv()
            rd["cw", 1, c] = make_chunk(1, c, ccw=False)
            rd["ccw", 0, c].wait_recv()
            rd["ccw", 1, c] = make_chunk(1, c, ccw=True)

        acc = acc + compute_block(wcw_ref[0])

        for c in range(2):
            rd["cw", 1, c].wait_recv()
            rd["cw", 2, c] = make_chunk(2, c, ccw=False)

        acc = acc + compute_block(wccw_ref[0])

        for c in range(2):
            rd["ccw", 1, c].wait_recv()
            rd["ccw", 2, c] = make_chunk(2, c, ccw=True)

        acc = acc + compute_block(wcw_ref[1])
        acc = acc + compute_block(wccw_ref[1])

        rd["cw", 2, 0].wait_recv()
        rd["cw", 2, 1].wait_recv()
        acc = acc + compute_block(wcw_ref[2])
        rd["ccw", 2, 0].wait_recv()
        rd["ccw", 2, 1].wait_recv()
        acc = acc + compute_block(wccw_ref[2])

        for key in rd:
            rd[key].wait_send()

        out_ref[...] = acc.reshape(Bl, Sq, D)

    return pl.pallas_call(
        body,
        out_shape=jax.ShapeDtypeStruct((Bl, Sq, D), jnp.float32),
        in_specs=[pl.BlockSpec(memory_space=pltpu.VMEM)] * 5,
        out_specs=pl.BlockSpec(memory_space=pltpu.VMEM),
        scratch_shapes=[
            pltpu.VMEM((NH, 4 * D, HW), jnp.bfloat16),
            pltpu.VMEM((NH, 4 * D, HW), jnp.bfloat16),
            pltpu.VMEM((BS, HD), jnp.bfloat16),
            pltpu.SemaphoreType.DMA((NH, 2)),
            pltpu.SemaphoreType.DMA((NH, 2)),
            pltpu.SemaphoreType.DMA((NH, 2)),
            pltpu.SemaphoreType.DMA((NH, 2)),
        ],
        compiler_params=pltpu.CompilerParams(collective_id=0),
    )(x, wcat, cos_t, sin_t, P_m)
